# baseline (speedup 1.0000x reference)
import numpy as np
import jax
import jax.numpy as jnp
from jax.experimental import pallas as pl
from jax.experimental.pallas import tpu as pltpu

H = 224
W = 224
H3 = 3 * H
W3 = 3 * W


def _stencil_matrix(n: int) -> np.ndarray:
    def refl(i):
        if i < 0:
            return -i
        if i >= n:
            return 2 * n - 2 - i
        return i

    s = np.zeros((n, 3 * n), np.float32)
    for J in range(3 * n):
        s[refl((J - 1) // 3), J] += 0.5
        s[refl((J + 1) // 3), J] += 0.5
    return s


def _rf_scale_kernel(x_ref, a_ref, s_ref, o_hbm, buf, sem):
    c = pl.program_id(0)
    nc = pl.num_programs(0)
    slot = c % 4

    @pl.when(c >= 4)
    def _():
        pltpu.make_async_copy(buf.at[slot], o_hbm.at[0, c - 4],
                              sem.at[slot]).wait()

    xb = x_ref[0, 0].astype(jnp.bfloat16)
    m1 = jnp.dot(xb, s_ref[...], preferred_element_type=jnp.float32)
    buf[slot] = jnp.dot(a_ref[...], m1.astype(jnp.bfloat16),
                        preferred_element_type=jnp.float32)

    pltpu.make_async_copy(buf.at[slot], o_hbm.at[0, c], sem.at[slot]).start()

    @pl.when(c == nc - 1)
    def _():
        for k in range(1, 4):
            pltpu.make_async_copy(buf.at[(slot - k) % 4], o_hbm.at[0, c - k],
                                  sem.at[(slot - k) % 4]).wait()
        pltpu.make_async_copy(buf.at[slot], o_hbm.at[0, c],
                              sem.at[slot]).wait()


def kernel(x):
    b, ch, h, w = x.shape
    s = jnp.asarray(_stencil_matrix(W), dtype=jnp.bfloat16)
    a = jnp.asarray(_stencil_matrix(H).T.copy(), dtype=jnp.bfloat16)
    out = pl.pallas_call(
        _rf_scale_kernel,
        grid=(ch,),
        in_specs=[
            pl.BlockSpec((1, 1, H, W), lambda c: (0, c, 0, 0)),
            pl.BlockSpec((H3, H), lambda c: (0, 0)),
            pl.BlockSpec((W, W3), lambda c: (0, 0)),
        ],
        out_specs=pl.BlockSpec(memory_space=pl.ANY),
        out_shape=jax.ShapeDtypeStruct((1, ch, H3, W3), x.dtype),
        scratch_shapes=[
            pltpu.VMEM((4, H3, W3), jnp.float32),
            pltpu.SemaphoreType.DMA((4,)),
        ],
        compiler_params=pltpu.CompilerParams(
            dimension_semantics=("arbitrary",)),
    )(x, a, s)
    return out


# speedup vs baseline: 1.3629x; 1.3629x over previous
import numpy as np
import jax
import jax.numpy as jnp
from jax.experimental import pallas as pl
from jax.experimental.pallas import tpu as pltpu

H = 224
W = 224
H3 = 3 * H
W3 = 3 * W
CB = 2
NSLOT = 4


def _stencil_matrix(n: int) -> np.ndarray:
    def refl(i):
        if i < 0:
            return -i
        if i >= n:
            return 2 * n - 2 - i
        return i

    s = np.zeros((n, 3 * n), np.float32)
    for J in range(3 * n):
        s[refl((J - 1) // 3), J] += 0.5
        s[refl((J + 1) // 3), J] += 0.5
    return s


def _rf_scale_kernel(x_ref, a_ref, s_ref, o_hbm, buf, sem):
    c = pl.program_id(0)
    nc = pl.num_programs(0)
    slot = c % NSLOT

    @pl.when(c >= NSLOT)
    def _():
        pltpu.make_async_copy(buf.at[slot],
                              o_hbm.at[0, pl.ds((c - NSLOT) * CB, CB)],
                              sem.at[slot]).wait()

    xb = x_ref[0].astype(jnp.bfloat16).reshape(CB * H, W)
    m1 = jnp.dot(xb, s_ref[...],
                 preferred_element_type=jnp.float32).astype(jnp.bfloat16)
    for k in range(CB):
        buf[slot, k] = jnp.dot(a_ref[...], m1[k * H:(k + 1) * H, :],
                               preferred_element_type=jnp.float32)

    pltpu.make_async_copy(buf.at[slot], o_hbm.at[0, pl.ds(c * CB, CB)],
                          sem.at[slot]).start()

    @pl.when(c == nc - 1)
    def _():
        for k in range(1, NSLOT):
            pltpu.make_async_copy(buf.at[(slot - k) % NSLOT],
                                  o_hbm.at[0, pl.ds((c - k) * CB, CB)],
                                  sem.at[(slot - k) % NSLOT]).wait()
        pltpu.make_async_copy(buf.at[slot], o_hbm.at[0, pl.ds(c * CB, CB)],
                              sem.at[slot]).wait()


def kernel(x):
    b, ch, h, w = x.shape
    s = jnp.asarray(_stencil_matrix(W), dtype=jnp.bfloat16)
    a = jnp.asarray(_stencil_matrix(H).T.copy(), dtype=jnp.bfloat16)
    out = pl.pallas_call(
        _rf_scale_kernel,
        grid=(ch // CB,),
        in_specs=[
            pl.BlockSpec((1, CB, H, W), lambda c: (0, c, 0, 0)),
            pl.BlockSpec((H3, H), lambda c: (0, 0)),
            pl.BlockSpec((W, W3), lambda c: (0, 0)),
        ],
        out_specs=pl.BlockSpec(memory_space=pl.ANY),
        out_shape=jax.ShapeDtypeStruct((1, ch, H3, W3), x.dtype),
        scratch_shapes=[
            pltpu.VMEM((NSLOT, CB, H3, W3), jnp.float32),
            pltpu.SemaphoreType.DMA((NSLOT,)),
        ],
        compiler_params=pltpu.CompilerParams(
            dimension_semantics=("arbitrary",)),
    )(x, a, s)
    return out


# speedup vs baseline: 1.4733x; 1.0811x over previous
import numpy as np
import jax
import jax.numpy as jnp
from jax.experimental import pallas as pl
from jax.experimental.pallas import tpu as pltpu

H = 224
W = 224
H3 = 3 * H
W3 = 3 * W
CB = 4
NSLOT = 3


def _stencil_matrix(n: int) -> np.ndarray:
    def refl(i):
        if i < 0:
            return -i
        if i >= n:
            return 2 * n - 2 - i
        return i

    s = np.zeros((n, 3 * n), np.float32)
    for J in range(3 * n):
        s[refl((J - 1) // 3), J] += 0.5
        s[refl((J + 1) // 3), J] += 0.5
    return s


def _rf_scale_kernel(x_ref, a_ref, s_ref, o_hbm, buf, sem):
    c = pl.program_id(0)
    nc = pl.num_programs(0)
    slot = c % NSLOT

    @pl.when(c >= NSLOT)
    def _():
        pltpu.make_async_copy(buf.at[slot],
                              o_hbm.at[0, pl.ds((c - NSLOT) * CB, CB)],
                              sem.at[slot]).wait()

    xb = x_ref[0].astype(jnp.bfloat16).reshape(CB * H, W)
    m1 = jnp.dot(xb, s_ref[...],
                 preferred_element_type=jnp.float32).astype(jnp.bfloat16)
    for k in range(CB):
        buf[slot, k] = jnp.dot(a_ref[...], m1[k * H:(k + 1) * H, :],
                               preferred_element_type=jnp.float32)

    pltpu.make_async_copy(buf.at[slot], o_hbm.at[0, pl.ds(c * CB, CB)],
                          sem.at[slot]).start()

    @pl.when(c == nc - 1)
    def _():
        for k in range(1, NSLOT):
            pltpu.make_async_copy(buf.at[(slot - k) % NSLOT],
                                  o_hbm.at[0, pl.ds((c - k) * CB, CB)],
                                  sem.at[(slot - k) % NSLOT]).wait()
        pltpu.make_async_copy(buf.at[slot], o_hbm.at[0, pl.ds(c * CB, CB)],
                              sem.at[slot]).wait()


def kernel(x):
    b, ch, h, w = x.shape
    s = jnp.asarray(_stencil_matrix(W), dtype=jnp.bfloat16)
    a = jnp.asarray(_stencil_matrix(H).T.copy(), dtype=jnp.bfloat16)
    out = pl.pallas_call(
        _rf_scale_kernel,
        grid=(ch // CB,),
        in_specs=[
            pl.BlockSpec((1, CB, H, W), lambda c: (0, c, 0, 0)),
            pl.BlockSpec((H3, H), lambda c: (0, 0)),
            pl.BlockSpec((W, W3), lambda c: (0, 0)),
        ],
        out_specs=pl.BlockSpec(memory_space=pl.ANY),
        out_shape=jax.ShapeDtypeStruct((1, ch, H3, W3), x.dtype),
        scratch_shapes=[
            pltpu.VMEM((NSLOT, CB, H3, W3), jnp.float32),
            pltpu.SemaphoreType.DMA((NSLOT,)),
        ],
        compiler_params=pltpu.CompilerParams(
            dimension_semantics=("arbitrary",)),
    )(x, a, s)
    return out


# speedup vs baseline: 1.4817x; 1.0057x over previous
import numpy as np
import jax
import jax.numpy as jnp
from jax.experimental import pallas as pl
from jax.experimental.pallas import tpu as pltpu

H = 224
W = 224
H3 = 3 * H
W3 = 3 * W
CB = 8
NSLOT = 2


def _stencil_matrix(n: int) -> np.ndarray:
    def refl(i):
        if i < 0:
            return -i
        if i >= n:
            return 2 * n - 2 - i
        return i

    s = np.zeros((n, 3 * n), np.float32)
    for J in range(3 * n):
        s[refl((J - 1) // 3), J] += 0.5
        s[refl((J + 1) // 3), J] += 0.5
    return s


def _rf_scale_kernel(x_ref, a_ref, s_ref, o_hbm, buf, sem):
    c = pl.program_id(0)
    nc = pl.num_programs(0)
    slot = c % NSLOT

    @pl.when(c >= NSLOT)
    def _():
        pltpu.make_async_copy(buf.at[slot],
                              o_hbm.at[0, pl.ds((c - NSLOT) * CB, CB)],
                              sem.at[slot]).wait()

    xb = x_ref[0].astype(jnp.bfloat16).reshape(CB * H, W)
    m1 = jnp.dot(xb, s_ref[...],
                 preferred_element_type=jnp.float32).astype(jnp.bfloat16)
    for k in range(CB):
        buf[slot, k] = jnp.dot(a_ref[...], m1[k * H:(k + 1) * H, :],
                               preferred_element_type=jnp.float32)

    pltpu.make_async_copy(buf.at[slot], o_hbm.at[0, pl.ds(c * CB, CB)],
                          sem.at[slot]).start()

    @pl.when(c == nc - 1)
    def _():
        for k in range(1, NSLOT):
            pltpu.make_async_copy(buf.at[(slot - k) % NSLOT],
                                  o_hbm.at[0, pl.ds((c - k) * CB, CB)],
                                  sem.at[(slot - k) % NSLOT]).wait()
        pltpu.make_async_copy(buf.at[slot], o_hbm.at[0, pl.ds(c * CB, CB)],
                              sem.at[slot]).wait()


def kernel(x):
    b, ch, h, w = x.shape
    s = jnp.asarray(_stencil_matrix(W), dtype=jnp.bfloat16)
    a = jnp.asarray(_stencil_matrix(H).T.copy(), dtype=jnp.bfloat16)
    out = pl.pallas_call(
        _rf_scale_kernel,
        grid=(ch // CB,),
        in_specs=[
            pl.BlockSpec((1, CB, H, W), lambda c: (0, c, 0, 0)),
            pl.BlockSpec((H3, H), lambda c: (0, 0)),
            pl.BlockSpec((W, W3), lambda c: (0, 0)),
        ],
        out_specs=pl.BlockSpec(memory_space=pl.ANY),
        out_shape=jax.ShapeDtypeStruct((1, ch, H3, W3), x.dtype),
        scratch_shapes=[
            pltpu.VMEM((NSLOT, CB, H3, W3), jnp.float32),
            pltpu.SemaphoreType.DMA((NSLOT,)),
        ],
        compiler_params=pltpu.CompilerParams(
            dimension_semantics=("arbitrary",)),
    )(x, a, s)
    return out
